# baseline (speedup 1.0000x reference)
import functools

import jax
import jax.numpy as jnp
from jax import lax
from jax.experimental import pallas as pl
from jax.experimental.pallas import tpu as pltpu
from jax.experimental.pallas import tpu_sc as plsc

N = 10000
D_IN = 128
H1, C1 = 8, 8
H2, C2 = 1, 7

NC, NS, LANES = 2, 16, 16
NW = NC * NS
K = 128
B = 4
N_PAD = 10240
RPS = N_PAD // NS

W72 = H1 * C1 + H1
W8 = H1
W2_8 = C2 + 1

_MESH = plsc.VectorSubcoreMesh(
    core_axis_name="c", subcore_axis_name="s", num_cores=NC, num_subcores=NS)
_SC_PARAMS = pltpu.CompilerParams(
    needs_layout_passes=False, use_tc_tiling_on_sc=False)


def _sc_edge_pass_l1(table72, tableD, idx, zeros72, nouter, e_total):
    T = nouter * B * K

    @functools.partial(
        pl.kernel,
        out_type=jax.ShapeDtypeStruct((NC, N_PAD, W72), jnp.float32),
        mesh=_MESH,
        compiler_params=_SC_PARAMS,
        scratch_types=[
            pltpu.VMEM_SHARED((N_PAD, W72), jnp.float32),
            pltpu.VMEM_SHARED((N, W8), jnp.float32),
            pltpu.VMEM((2 * B, K), jnp.int32),
            pltpu.VMEM((B * K, W72), jnp.float32),
            pltpu.VMEM((B * K, W8), jnp.float32),
            pltpu.VMEM((B * K, W72), jnp.float32),
            pltpu.SemaphoreType.DMA,
            pltpu.SemaphoreType.DMA,
            pltpu.SemaphoreType.DMA,
        ],
    )
    def k(t72_hbm, tD_hbm, idx_hbm, z_hbm, part_hbm,
          accum, adshared, idxv, rowsv, rowsdv, outv, gsem, g2sem, ssem):
        cid = lax.axis_index("c")
        sid = lax.axis_index("s")
        wid = sid * NC + cid
        r0 = sid * RPS
        pltpu.sync_copy(z_hbm.at[pl.ds(r0, RPS)], accum.at[pl.ds(r0, RPS)])
        d0 = sid * (N // NS)
        pltpu.sync_copy(tD_hbm.at[pl.ds(d0, N // NS)], adshared.at[pl.ds(d0, N // NS)])
        plsc.subcore_barrier()

        iota = lax.iota(jnp.int32, LANES)

        def outer(it, carry):
            r = (wid * nouter + it) * (2 * B)
            pltpu.sync_copy(idx_hbm.at[pl.ds(r, 2 * B)], idxv)
            descs = []
            for j in range(B):
                descs.append(pltpu.async_copy(
                    t72_hbm.at[idxv.at[j]], rowsv.at[pl.ds(j * K, K)], gsem))
                descs.append(pltpu.async_copy(
                    adshared.at[idxv.at[B + j]], rowsdv.at[pl.ds(j * K, K)], g2sem))
            for d in descs:
                d.wait()

            ebase = wid * T + it * (B * K)

            def group(g, carry2):
                rows = g * LANES + iota
                fmask = jnp.where((ebase + g * LANES + iota) < e_total, 1.0, 0.0)
                for hd in range(H1):
                    colw = jnp.full((LANES,), H1 * C1 + hd, jnp.int32)
                    a_s = plsc.load_gather(rowsv, [rows, colw])
                    a_d = plsc.load_gather(rowsdv, [rows, jnp.full((LANES,), hd, jnp.int32)])
                    alpha = a_s + a_d
                    alpha = jnp.where(alpha >= 0, alpha, 0.2 * alpha)
                    w = jnp.exp(alpha) * fmask
                    plsc.store_scatter(outv, [rows, colw], w)
                    for c in range(C1):
                        col = jnp.full((LANES,), hd * C1 + c, jnp.int32)
                        hv = plsc.load_gather(rowsv, [rows, col])
                        plsc.store_scatter(outv, [rows, col], w * hv)
                return carry2

            lax.fori_loop(0, B * K // LANES, group, 0)

            sdescs = [pltpu.async_copy(
                outv.at[pl.ds(j * K, K)], accum.at[idxv.at[B + j]], ssem, add=True)
                for j in range(B)]
            for d in sdescs:
                d.wait()
            return carry

        lax.fori_loop(0, nouter, outer, 0)
        plsc.subcore_barrier()
        pltpu.sync_copy(accum.at[pl.ds(r0, RPS)], part_hbm.at[cid, pl.ds(r0, RPS)])

    return k(table72, tableD, idx, zeros72)


def _sc_edge_pass_l2(table8, ad2, idx, zeros8, nouter, e_total):
    T = nouter * B * K

    @functools.partial(
        pl.kernel,
        out_type=jax.ShapeDtypeStruct((NC, N_PAD, W2_8), jnp.float32),
        mesh=_MESH,
        compiler_params=_SC_PARAMS,
        scratch_types=[
            pltpu.VMEM_SHARED((N_PAD, W2_8), jnp.float32),
            pltpu.VMEM((N,), jnp.float32),
            pltpu.VMEM((2 * B, K), jnp.int32),
            pltpu.VMEM((B * K, W2_8), jnp.float32),
            pltpu.VMEM((B * K, W2_8), jnp.float32),
            pltpu.SemaphoreType.DMA,
            pltpu.SemaphoreType.DMA,
        ],
    )
    def k(t8_hbm, ad_hbm, idx_hbm, z_hbm, part_hbm,
          accum, adbuf, idxv, rowsv, outv, gsem, ssem):
        cid = lax.axis_index("c")
        sid = lax.axis_index("s")
        wid = sid * NC + cid
        r0 = sid * RPS
        pltpu.sync_copy(z_hbm.at[pl.ds(r0, RPS)], accum.at[pl.ds(r0, RPS)])
        pltpu.sync_copy(ad_hbm, adbuf)
        plsc.subcore_barrier()

        iota = lax.iota(jnp.int32, LANES)

        def outer(it, carry):
            r = (wid * nouter + it) * (2 * B)
            pltpu.sync_copy(idx_hbm.at[pl.ds(r, 2 * B)], idxv)
            descs = [pltpu.async_copy(
                t8_hbm.at[idxv.at[j]], rowsv.at[pl.ds(j * K, K)], gsem)
                for j in range(B)]
            for d in descs:
                d.wait()

            ebase = wid * T + it * (B * K)

            def group(g, carry2):
                rows = g * LANES + iota
                fmask = jnp.where((ebase + g * LANES + iota) < e_total, 1.0, 0.0)
                j = g // (K // LANES)
                gk = g % (K // LANES)
                dvals = idxv[B + j, pl.ds(gk * LANES, LANES)]
                colw = jnp.full((LANES,), C2, jnp.int32)
                a_s = plsc.load_gather(rowsv, [rows, colw])
                a_d = plsc.load_gather(adbuf, [dvals])
                alpha = a_s + a_d
                alpha = jnp.where(alpha >= 0, alpha, 0.2 * alpha)
                w = jnp.exp(alpha) * fmask
                plsc.store_scatter(outv, [rows, colw], w)
                for c in range(C2):
                    col = jnp.full((LANES,), c, jnp.int32)
                    hv = plsc.load_gather(rowsv, [rows, col])
                    plsc.store_scatter(outv, [rows, col], w * hv)
                return carry2

            lax.fori_loop(0, B * K // LANES, group, 0)

            sdescs = [pltpu.async_copy(
                outv.at[pl.ds(j * K, K)], accum.at[idxv.at[B + j]], ssem, add=True)
                for j in range(B)]
            for d in sdescs:
                d.wait()
            return carry

        lax.fori_loop(0, nouter, outer, 0)
        plsc.subcore_barrier()
        pltpu.sync_copy(accum.at[pl.ds(r0, RPS)], part_hbm.at[cid, pl.ds(r0, RPS)])

    return k(table8, ad2, idx, zeros8)


_BN = 1000


def _tc_tables1(x, wcat):
    def body(x_ref, w_ref, o72_ref, o8_ref):
        h = jnp.dot(x_ref[...], w_ref[...], preferred_element_type=jnp.float32)
        o72_ref[...] = h[:, :W72]
        o8_ref[...] = h[:, W72:]

    return pl.pallas_call(
        body,
        grid=(N // _BN,),
        in_specs=[pl.BlockSpec((_BN, D_IN), lambda i: (i, 0)),
                  pl.BlockSpec((D_IN, W72 + W8), lambda i: (0, 0))],
        out_specs=[pl.BlockSpec((_BN, W72), lambda i: (i, 0)),
                   pl.BlockSpec((_BN, W8), lambda i: (i, 0))],
        out_shape=[jax.ShapeDtypeStruct((N, W72), jnp.float32),
                   jax.ShapeDtypeStruct((N, W8), jnp.float32)],
    )(x, wcat)


def _tc_mid(part1, b1row, rrep, m8, adv):
    def body(p_ref, b_ref, r_ref, m_ref, a_ref, t8_ref, ad_ref):
        num = p_ref[0, :, :H1 * C1] + p_ref[1, :, :H1 * C1]
        den = p_ref[0, :, H1 * C1:] + p_ref[1, :, H1 * C1:]
        denr = jnp.dot(den, r_ref[...], preferred_element_type=jnp.float32)
        out1 = num / denr + b_ref[...]
        t8_ref[...] = jnp.dot(out1, m_ref[...], preferred_element_type=jnp.float32)
        ad_ref[...] = jnp.dot(out1, a_ref[...], preferred_element_type=jnp.float32)

    return pl.pallas_call(
        body,
        grid=(N // _BN,),
        in_specs=[pl.BlockSpec((NC, _BN, W72), lambda i: (0, i, 0)),
                  pl.BlockSpec((1, H1 * C1), lambda i: (0, 0)),
                  pl.BlockSpec((H1, H1 * C1), lambda i: (0, 0)),
                  pl.BlockSpec((H1 * C1, W2_8), lambda i: (0, 0)),
                  pl.BlockSpec((H1 * C1, 1), lambda i: (0, 0))],
        out_specs=[pl.BlockSpec((_BN, W2_8), lambda i: (i, 0)),
                   pl.BlockSpec((_BN, 1), lambda i: (i, 0))],
        out_shape=[jax.ShapeDtypeStruct((N, W2_8), jnp.float32),
                   jax.ShapeDtypeStruct((N, 1), jnp.float32)],
    )(part1, b1row, rrep, m8, adv)


def _tc_final(part2, b2row):
    def body(p_ref, b_ref, o_ref):
        num = p_ref[0, :, :C2] + p_ref[1, :, :C2]
        den = p_ref[0, :, C2:] + p_ref[1, :, C2:]
        o = num / den + b_ref[...]
        m = jnp.max(o, axis=1, keepdims=True)
        ex = jnp.exp(o - m)
        o_ref[...] = (o - m) - jnp.log(jnp.sum(ex, axis=1, keepdims=True))

    return pl.pallas_call(
        body,
        grid=(N // _BN,),
        in_specs=[pl.BlockSpec((NC, _BN, W2_8), lambda i: (0, i, 0)),
                  pl.BlockSpec((1, C2), lambda i: (0, 0))],
        out_specs=pl.BlockSpec((_BN, C2), lambda i: (i, 0)),
        out_shape=jax.ShapeDtypeStruct((N, C2), jnp.float32),
    )(part2, b2row)


def kernel(x, edge_index, W1, att_src1, att_dst1, b1, W2, att_src2, att_dst2, b2):
    W1r = W1.reshape(D_IN, H1, C1)
    wsrc1 = jnp.einsum("dhc,hc->dh", W1r, att_src1[0])
    wdst1 = jnp.einsum("dhc,hc->dh", W1r, att_dst1[0])
    wcat = jnp.concatenate([W1, wsrc1, wdst1], axis=1)

    m8 = jnp.concatenate([W2, (W2 @ att_src2[0, 0])[:, None]], axis=1)
    adv = (W2 @ att_dst2[0, 0])[:, None]
    rrep = jnp.repeat(jnp.eye(H1, dtype=jnp.float32), C1, axis=1)
    b1row = b1.reshape(1, H1 * C1)
    b2row = b2.reshape(1, C2)

    e_in = edge_index.shape[1]
    e_total = e_in + N
    nouter = -(-e_total // (NW * B * K))
    e_pad = NW * nouter * B * K
    loops = jnp.arange(N, dtype=jnp.int32)
    padz = jnp.zeros((e_pad - e_total,), jnp.int32)
    src = jnp.concatenate([edge_index[0].astype(jnp.int32), loops, padz])
    dst = jnp.concatenate([edge_index[1].astype(jnp.int32), loops, padz])
    idx = jnp.stack([src.reshape(NW, nouter, B, K),
                     dst.reshape(NW, nouter, B, K)], axis=2)
    idx = idx.reshape(NW * nouter * 2 * B, K)

    zeros72 = jnp.zeros((N_PAD, W72), jnp.float32)
    zeros8 = jnp.zeros((N_PAD, W2_8), jnp.float32)

    table72, tableD = _tc_tables1(x, wcat)
    part1 = _sc_edge_pass_l1(table72, tableD, idx, zeros72, nouter, e_total)
    table8, ad2 = _tc_mid(part1, b1row, rrep, m8, adv)
    part2 = _sc_edge_pass_l2(table8, ad2.reshape(N), idx, zeros8, nouter, e_total)
    return _tc_final(part2, b2row)


# speedup vs baseline: 1.1263x; 1.1263x over previous
import functools

import jax
import jax.numpy as jnp
from jax import lax
from jax.experimental import pallas as pl
from jax.experimental.pallas import tpu as pltpu
from jax.experimental.pallas import tpu_sc as plsc

N = 10000
D_IN = 128
H1, C1 = 8, 8
H2, C2 = 1, 7

NC, NS, LANES = 2, 16, 16
NW = NC * NS
K = 128
B = 2
N_PAD = 10240
RPS = N_PAD // NS

W72 = H1 * C1 + H1
W8 = H1
W2_8 = C2 + 1

_MESH = plsc.VectorSubcoreMesh(
    core_axis_name="c", subcore_axis_name="s", num_cores=NC, num_subcores=NS)
_SC_PARAMS = pltpu.CompilerParams(
    needs_layout_passes=False, use_tc_tiling_on_sc=False)


def _sc_edge_pass_l1(table72, tableD, idx, zeros72, nouter, e_total):
    T = nouter * B * K

    @functools.partial(
        pl.kernel,
        out_type=jax.ShapeDtypeStruct((NC, N_PAD, W72), jnp.float32),
        mesh=_MESH,
        compiler_params=_SC_PARAMS,
        scratch_types=[
            pltpu.VMEM_SHARED((N_PAD, W72), jnp.float32),
            pltpu.VMEM_SHARED((N, W8), jnp.float32),
            [pltpu.VMEM((2 * B, K), jnp.int32)] * 2,
            [pltpu.VMEM((B * K, W72), jnp.float32)] * 2,
            [pltpu.VMEM((B * K, W8), jnp.float32)] * 2,
            [pltpu.VMEM((B * K, W72), jnp.float32)] * 2,
            pltpu.SemaphoreType.DMA,
            pltpu.SemaphoreType.DMA,
            pltpu.SemaphoreType.DMA,
        ],
    )
    def k(t72_hbm, tD_hbm, idx_hbm, z_hbm, part_hbm,
          accum, adshared, idxv, rowsv, rowsdv, outv, gsem, g2sem, ssem):
        cid = lax.axis_index("c")
        sid = lax.axis_index("s")
        wid = sid * NC + cid
        r0 = sid * RPS
        pltpu.sync_copy(z_hbm.at[pl.ds(r0, RPS)], accum.at[pl.ds(r0, RPS)])
        d0 = sid * (N // NS)
        pltpu.sync_copy(tD_hbm.at[pl.ds(d0, N // NS)], adshared.at[pl.ds(d0, N // NS)])
        plsc.subcore_barrier()

        iota = lax.iota(jnp.int32, LANES)

        def load_idx(it, buf):
            r = (wid * (nouter + 1) + it) * (2 * B)
            pltpu.sync_copy(idx_hbm.at[pl.ds(r, 2 * B)], idxv[buf])

        def fire_gathers(buf):
            ds = []
            for j in range(B):
                ds.append(pltpu.async_copy(
                    t72_hbm.at[idxv[buf].at[j]],
                    rowsv[buf].at[pl.ds(j * K, K)], gsem))
                ds.append(pltpu.async_copy(
                    adshared.at[idxv[buf].at[B + j]],
                    rowsdv[buf].at[pl.ds(j * K, K)], g2sem))
            return ds

        def compute(it, buf):
            rv, rdv, ov = rowsv[buf], rowsdv[buf], outv[buf]
            ebase = wid * T + it * (B * K)

            def group(g, carry2):
                rows = g * LANES + iota
                fmask = jnp.where((ebase + g * LANES + iota) < e_total, 1.0, 0.0)
                for hd in range(H1):
                    colw = jnp.full((LANES,), H1 * C1 + hd, jnp.int32)
                    a_s = plsc.load_gather(rv, [rows, colw])
                    a_d = plsc.load_gather(rdv, [rows, jnp.full((LANES,), hd, jnp.int32)])
                    alpha = a_s + a_d
                    alpha = jnp.where(alpha >= 0, alpha, 0.2 * alpha)
                    w = jnp.exp(alpha) * fmask
                    plsc.store_scatter(ov, [rows, colw], w)
                    for c in range(C1):
                        col = jnp.full((LANES,), hd * C1 + c, jnp.int32)
                        hv = plsc.load_gather(rv, [rows, col])
                        plsc.store_scatter(ov, [rows, col], w * hv)
                return carry2

            lax.fori_loop(0, B * K // LANES, group, 0)

        def fire_scatters(buf):
            return [pltpu.async_copy(
                outv[buf].at[pl.ds(j * K, K)],
                accum.at[idxv[buf].at[B + j]], ssem, add=True)
                for j in range(B)]

        load_idx(0, 0)
        for d in fire_gathers(0):
            d.wait()

        def pair(ip, carry):
            for cur in range(2):
                it = ip * 2 + cur
                nxt = 1 - cur
                load_idx(it + 1, nxt)
                gds = fire_gathers(nxt)
                compute(it, cur)
                sds = fire_scatters(cur)
                for d in gds:
                    d.wait()
                for d in sds:
                    d.wait()
            return carry

        lax.fori_loop(0, nouter // 2, pair, 0)
        plsc.subcore_barrier()
        pltpu.sync_copy(accum.at[pl.ds(r0, RPS)], part_hbm.at[cid, pl.ds(r0, RPS)])

    return k(table72, tableD, idx, zeros72)


def _sc_edge_pass_l2(table8, ad2, idx, zeros8, nouter, e_total):
    T = nouter * B * K

    @functools.partial(
        pl.kernel,
        out_type=jax.ShapeDtypeStruct((NC, N_PAD, W2_8), jnp.float32),
        mesh=_MESH,
        compiler_params=_SC_PARAMS,
        scratch_types=[
            pltpu.VMEM_SHARED((N_PAD, W2_8), jnp.float32),
            pltpu.VMEM((N,), jnp.float32),
            [pltpu.VMEM((2 * B, K), jnp.int32)] * 2,
            [pltpu.VMEM((B * K, W2_8), jnp.float32)] * 2,
            [pltpu.VMEM((B * K, W2_8), jnp.float32)] * 2,
            pltpu.SemaphoreType.DMA,
            pltpu.SemaphoreType.DMA,
        ],
    )
    def k(t8_hbm, ad_hbm, idx_hbm, z_hbm, part_hbm,
          accum, adbuf, idxv, rowsv, outv, gsem, ssem):
        cid = lax.axis_index("c")
        sid = lax.axis_index("s")
        wid = sid * NC + cid
        r0 = sid * RPS
        pltpu.sync_copy(z_hbm.at[pl.ds(r0, RPS)], accum.at[pl.ds(r0, RPS)])
        pltpu.sync_copy(ad_hbm, adbuf)
        plsc.subcore_barrier()

        iota = lax.iota(jnp.int32, LANES)

        def load_idx(it, buf):
            r = (wid * (nouter + 1) + it) * (2 * B)
            pltpu.sync_copy(idx_hbm.at[pl.ds(r, 2 * B)], idxv[buf])

        def fire_gathers(buf):
            return [pltpu.async_copy(
                t8_hbm.at[idxv[buf].at[j]],
                rowsv[buf].at[pl.ds(j * K, K)], gsem)
                for j in range(B)]

        def compute(it, buf):
            rv, ov, iv = rowsv[buf], outv[buf], idxv[buf]
            ebase = wid * T + it * (B * K)

            def group(g, carry2):
                rows = g * LANES + iota
                fmask = jnp.where((ebase + g * LANES + iota) < e_total, 1.0, 0.0)
                j = g // (K // LANES)
                gk = g % (K // LANES)
                dvals = iv[B + j, pl.ds(gk * LANES, LANES)]
                colw = jnp.full((LANES,), C2, jnp.int32)
                a_s = plsc.load_gather(rv, [rows, colw])
                a_d = plsc.load_gather(adbuf, [dvals])
                alpha = a_s + a_d
                alpha = jnp.where(alpha >= 0, alpha, 0.2 * alpha)
                w = jnp.exp(alpha) * fmask
                plsc.store_scatter(ov, [rows, colw], w)
                for c in range(C2):
                    col = jnp.full((LANES,), c, jnp.int32)
                    hv = plsc.load_gather(rv, [rows, col])
                    plsc.store_scatter(ov, [rows, col], w * hv)
                return carry2

            lax.fori_loop(0, B * K // LANES, group, 0)

        def fire_scatters(buf):
            return [pltpu.async_copy(
                outv[buf].at[pl.ds(j * K, K)],
                accum.at[idxv[buf].at[B + j]], ssem, add=True)
                for j in range(B)]

        load_idx(0, 0)
        for d in fire_gathers(0):
            d.wait()

        def pair(ip, carry):
            for cur in range(2):
                it = ip * 2 + cur
                nxt = 1 - cur
                load_idx(it + 1, nxt)
                gds = fire_gathers(nxt)
                compute(it, cur)
                sds = fire_scatters(cur)
                for d in gds:
                    d.wait()
                for d in sds:
                    d.wait()
            return carry

        lax.fori_loop(0, nouter // 2, pair, 0)
        plsc.subcore_barrier()
        pltpu.sync_copy(accum.at[pl.ds(r0, RPS)], part_hbm.at[cid, pl.ds(r0, RPS)])

    return k(table8, ad2, idx, zeros8)


_BN = 1000


def _tc_tables1(x, wcat):
    def body(x_ref, w_ref, o72_ref, o8_ref):
        h = jnp.dot(x_ref[...], w_ref[...], preferred_element_type=jnp.float32)
        o72_ref[...] = h[:, :W72]
        o8_ref[...] = h[:, W72:]

    return pl.pallas_call(
        body,
        grid=(N // _BN,),
        in_specs=[pl.BlockSpec((_BN, D_IN), lambda i: (i, 0)),
                  pl.BlockSpec((D_IN, W72 + W8), lambda i: (0, 0))],
        out_specs=[pl.BlockSpec((_BN, W72), lambda i: (i, 0)),
                   pl.BlockSpec((_BN, W8), lambda i: (i, 0))],
        out_shape=[jax.ShapeDtypeStruct((N, W72), jnp.float32),
                   jax.ShapeDtypeStruct((N, W8), jnp.float32)],
    )(x, wcat)


def _tc_mid(part1, b1row, rrep, m8, adv):
    def body(p_ref, b_ref, r_ref, m_ref, a_ref, t8_ref, ad_ref):
        num = p_ref[0, :, :H1 * C1] + p_ref[1, :, :H1 * C1]
        den = p_ref[0, :, H1 * C1:] + p_ref[1, :, H1 * C1:]
        denr = jnp.dot(den, r_ref[...], preferred_element_type=jnp.float32)
        out1 = num / denr + b_ref[...]
        t8_ref[...] = jnp.dot(out1, m_ref[...], preferred_element_type=jnp.float32)
        ad_ref[...] = jnp.dot(out1, a_ref[...], preferred_element_type=jnp.float32)

    return pl.pallas_call(
        body,
        grid=(N // _BN,),
        in_specs=[pl.BlockSpec((NC, _BN, W72), lambda i: (0, i, 0)),
                  pl.BlockSpec((1, H1 * C1), lambda i: (0, 0)),
                  pl.BlockSpec((H1, H1 * C1), lambda i: (0, 0)),
                  pl.BlockSpec((H1 * C1, W2_8), lambda i: (0, 0)),
                  pl.BlockSpec((H1 * C1, 1), lambda i: (0, 0))],
        out_specs=[pl.BlockSpec((_BN, W2_8), lambda i: (i, 0)),
                   pl.BlockSpec((_BN, 1), lambda i: (i, 0))],
        out_shape=[jax.ShapeDtypeStruct((N, W2_8), jnp.float32),
                   jax.ShapeDtypeStruct((N, 1), jnp.float32)],
    )(part1, b1row, rrep, m8, adv)


def _tc_final(part2, b2row):
    def body(p_ref, b_ref, o_ref):
        num = p_ref[0, :, :C2] + p_ref[1, :, :C2]
        den = p_ref[0, :, C2:] + p_ref[1, :, C2:]
        o = num / den + b_ref[...]
        m = jnp.max(o, axis=1, keepdims=True)
        ex = jnp.exp(o - m)
        o_ref[...] = (o - m) - jnp.log(jnp.sum(ex, axis=1, keepdims=True))

    return pl.pallas_call(
        body,
        grid=(N // _BN,),
        in_specs=[pl.BlockSpec((NC, _BN, W2_8), lambda i: (0, i, 0)),
                  pl.BlockSpec((1, C2), lambda i: (0, 0))],
        out_specs=pl.BlockSpec((_BN, C2), lambda i: (i, 0)),
        out_shape=jax.ShapeDtypeStruct((N, C2), jnp.float32),
    )(part2, b2row)


def kernel(x, edge_index, W1, att_src1, att_dst1, b1, W2, att_src2, att_dst2, b2):
    W1r = W1.reshape(D_IN, H1, C1)
    wsrc1 = jnp.einsum("dhc,hc->dh", W1r, att_src1[0])
    wdst1 = jnp.einsum("dhc,hc->dh", W1r, att_dst1[0])
    wcat = jnp.concatenate([W1, wsrc1, wdst1], axis=1)

    m8 = jnp.concatenate([W2, (W2 @ att_src2[0, 0])[:, None]], axis=1)
    adv = (W2 @ att_dst2[0, 0])[:, None]
    rrep = jnp.repeat(jnp.eye(H1, dtype=jnp.float32), C1, axis=1)
    b1row = b1.reshape(1, H1 * C1)
    b2row = b2.reshape(1, C2)

    e_in = edge_index.shape[1]
    e_total = e_in + N
    nouter = -(-e_total // (NW * B * K))
    nouter += nouter % 2
    e_pad = NW * nouter * B * K
    loops = jnp.arange(N, dtype=jnp.int32)
    padz = jnp.zeros((e_pad - e_total,), jnp.int32)
    src = jnp.concatenate([edge_index[0].astype(jnp.int32), loops, padz])
    dst = jnp.concatenate([edge_index[1].astype(jnp.int32), loops, padz])
    idx = jnp.stack([src.reshape(NW, nouter, B, K),
                     dst.reshape(NW, nouter, B, K)], axis=2)
    idx = jnp.concatenate(
        [idx, jnp.zeros((NW, 1, 2, B, K), jnp.int32)], axis=1)
    idx = idx.reshape(NW * (nouter + 1) * 2 * B, K)

    zeros72 = jnp.zeros((N_PAD, W72), jnp.float32)
    zeros8 = jnp.zeros((N_PAD, W2_8), jnp.float32)

    table72, tableD = _tc_tables1(x, wcat)
    part1 = _sc_edge_pass_l1(table72, tableD, idx, zeros72, nouter, e_total)
    table8, ad2 = _tc_mid(part1, b1row, rrep, m8, adv)
    part2 = _sc_edge_pass_l2(table8, ad2.reshape(N), idx, zeros8, nouter, e_total)
    return _tc_final(part2, b2row)


# speedup vs baseline: 1.4269x; 1.2668x over previous
import functools

import jax
import jax.numpy as jnp
from jax import lax
from jax.experimental import pallas as pl
from jax.experimental.pallas import tpu as pltpu
from jax.experimental.pallas import tpu_sc as plsc

N = 10000
D_IN = 128
H1, C1 = 8, 8
H2, C2 = 1, 7

NC, NS, LANES = 2, 16, 16
NW = NC * NS
K = 128
B = 2
N_PAD = 10240
RPS = N_PAD // NS

HH = H1 // NC
W36 = HH * C1 + HH
W4 = HH
W2_8 = C2 + 1

_MESH = plsc.VectorSubcoreMesh(
    core_axis_name="c", subcore_axis_name="s", num_cores=NC, num_subcores=NS)
_SC_PARAMS = pltpu.CompilerParams(
    needs_layout_passes=False, use_tc_tiling_on_sc=False)


def _sc_edge_pass_l1(t36, tD, idx, zeros36, nouter, e_total):
    T = nouter * B * K

    @functools.partial(
        pl.kernel,
        out_type=jax.ShapeDtypeStruct((NC, N_PAD, W36), jnp.float32),
        mesh=_MESH,
        compiler_params=_SC_PARAMS,
        scratch_types=[
            pltpu.VMEM_SHARED((N_PAD, W36), jnp.float32),
            pltpu.VMEM_SHARED((N_PAD, W36), jnp.float32),
            pltpu.VMEM_SHARED((N_PAD, W4), jnp.float32),
            [pltpu.VMEM((2 * B, K), jnp.int32)] * 2,
            [pltpu.VMEM((B * K, W36), jnp.float32)] * 2,
            [pltpu.VMEM((B * K, W4), jnp.float32)] * 2,
            [pltpu.VMEM((B * K, W36), jnp.float32)] * 2,
            pltpu.SemaphoreType.DMA,
            pltpu.SemaphoreType.DMA,
            pltpu.SemaphoreType.DMA,
        ],
    )
    def k(t36_hbm, tD_hbm, idx_hbm, z_hbm, part_hbm,
          accum, tsh, adsh, idxv, rowsv, rowsdv, outv, gsem, g2sem, ssem):
        cid = lax.axis_index("c")
        sid = lax.axis_index("s")
        r0 = sid * RPS
        pltpu.sync_copy(z_hbm.at[pl.ds(r0, RPS)], accum.at[pl.ds(r0, RPS)])
        pltpu.sync_copy(t36_hbm.at[cid, pl.ds(r0, RPS)], tsh.at[pl.ds(r0, RPS)])
        pltpu.sync_copy(tD_hbm.at[cid, pl.ds(r0, RPS)], adsh.at[pl.ds(r0, RPS)])
        plsc.subcore_barrier()

        iota = lax.iota(jnp.int32, LANES)

        def load_idx(it, buf):
            r = (sid * (nouter + 1) + it) * (2 * B)
            pltpu.sync_copy(idx_hbm.at[pl.ds(r, 2 * B)], idxv[buf])

        def fire_gathers(buf):
            ds = []
            for j in range(B):
                ds.append(pltpu.async_copy(
                    tsh.at[idxv[buf].at[j]],
                    rowsv[buf].at[pl.ds(j * K, K)], gsem))
                ds.append(pltpu.async_copy(
                    adsh.at[idxv[buf].at[B + j]],
                    rowsdv[buf].at[pl.ds(j * K, K)], g2sem))
            return ds

        def compute(it, buf):
            rv, rdv, ov = rowsv[buf], rowsdv[buf], outv[buf]
            ebase = sid * T + it * (B * K)

            def group(g, carry2):
                rows = g * LANES + iota
                fmask = jnp.where((ebase + g * LANES + iota) < e_total, 1.0, 0.0)
                for hd in range(HH):
                    colw = jnp.full((LANES,), HH * C1 + hd, jnp.int32)
                    a_s = plsc.load_gather(rv, [rows, colw])
                    a_d = plsc.load_gather(rdv, [rows, jnp.full((LANES,), hd, jnp.int32)])
                    alpha = a_s + a_d
                    alpha = jnp.where(alpha >= 0, alpha, 0.2 * alpha)
                    w = jnp.exp(alpha) * fmask
                    plsc.store_scatter(ov, [rows, colw], w)
                    for c in range(C1):
                        col = jnp.full((LANES,), hd * C1 + c, jnp.int32)
                        hv = plsc.load_gather(rv, [rows, col])
                        plsc.store_scatter(ov, [rows, col], w * hv)
                return carry2

            lax.fori_loop(0, B * K // LANES, group, 0)

        def fire_scatters(buf):
            return [pltpu.async_copy(
                outv[buf].at[pl.ds(j * K, K)],
                accum.at[idxv[buf].at[B + j]], ssem, add=True)
                for j in range(B)]

        load_idx(0, 0)
        for d in fire_gathers(0):
            d.wait()

        def pair(ip, carry):
            for cur in range(2):
                it = ip * 2 + cur
                nxt = 1 - cur
                load_idx(it + 1, nxt)
                gds = fire_gathers(nxt)
                compute(it, cur)
                sds = fire_scatters(cur)
                for d in gds:
                    d.wait()
                for d in sds:
                    d.wait()
            return carry

        lax.fori_loop(0, nouter // 2, pair, 0)
        plsc.subcore_barrier()
        pltpu.sync_copy(accum.at[pl.ds(r0, RPS)], part_hbm.at[cid, pl.ds(r0, RPS)])

    return k(t36, tD, idx, zeros36)


def _sc_edge_pass_l2(table8, ad2, idx, zeros8, nouter, e_total):
    T = nouter * B * K

    @functools.partial(
        pl.kernel,
        out_type=jax.ShapeDtypeStruct((NC, N_PAD, W2_8), jnp.float32),
        mesh=_MESH,
        compiler_params=_SC_PARAMS,
        scratch_types=[
            pltpu.VMEM_SHARED((N_PAD, W2_8), jnp.float32),
            pltpu.VMEM((N, W2_8), jnp.float32),
            pltpu.VMEM((N,), jnp.float32),
            [pltpu.VMEM((2 * B, K), jnp.int32)] * 2,
            [pltpu.VMEM((B * K, W2_8), jnp.float32)] * 2,
            pltpu.SemaphoreType.DMA,
        ],
    )
    def k(t8_hbm, ad_hbm, idx_hbm, z_hbm, part_hbm,
          accum, t8buf, adbuf, idxv, outv, ssem):
        cid = lax.axis_index("c")
        sid = lax.axis_index("s")
        wid = sid * NC + cid
        r0 = sid * RPS
        pltpu.sync_copy(z_hbm.at[pl.ds(r0, RPS)], accum.at[pl.ds(r0, RPS)])
        pltpu.sync_copy(t8_hbm, t8buf)
        pltpu.sync_copy(ad_hbm, adbuf)
        plsc.subcore_barrier()

        iota = lax.iota(jnp.int32, LANES)

        def load_idx(it, buf):
            r = (wid * (nouter + 1) + it) * (2 * B)
            pltpu.sync_copy(idx_hbm.at[pl.ds(r, 2 * B)], idxv[buf])

        def compute(it, buf):
            ov, iv = outv[buf], idxv[buf]
            ebase = wid * T + it * (B * K)

            def group(g, carry2):
                rows = g * LANES + iota
                fmask = jnp.where((ebase + g * LANES + iota) < e_total, 1.0, 0.0)
                j = g // (K // LANES)
                gk = g % (K // LANES)
                svals = iv[j, pl.ds(gk * LANES, LANES)]
                dvals = iv[B + j, pl.ds(gk * LANES, LANES)]
                colw = jnp.full((LANES,), C2, jnp.int32)
                a_s = plsc.load_gather(t8buf, [svals, colw])
                a_d = plsc.load_gather(adbuf, [dvals])
                alpha = a_s + a_d
                alpha = jnp.where(alpha >= 0, alpha, 0.2 * alpha)
                w = jnp.exp(alpha) * fmask
                plsc.store_scatter(ov, [rows, colw], w)
                for c in range(C2):
                    col = jnp.full((LANES,), c, jnp.int32)
                    hv = plsc.load_gather(t8buf, [svals, col])
                    plsc.store_scatter(ov, [rows, col], w * hv)
                return carry2

            lax.fori_loop(0, B * K // LANES, group, 0)

        def fire_scatters(buf):
            return [pltpu.async_copy(
                outv[buf].at[pl.ds(j * K, K)],
                accum.at[idxv[buf].at[B + j]], ssem, add=True)
                for j in range(B)]

        load_idx(0, 0)

        def pair(ip, carry):
            for cur in range(2):
                it = ip * 2 + cur
                nxt = 1 - cur
                load_idx(it + 1, nxt)
                compute(it, cur)
                sds = fire_scatters(cur)
                for d in sds:
                    d.wait()
            return carry

        lax.fori_loop(0, nouter // 2, pair, 0)
        plsc.subcore_barrier()
        pltpu.sync_copy(accum.at[pl.ds(r0, RPS)], part_hbm.at[cid, pl.ds(r0, RPS)])

    return k(table8, ad2, idx, zeros8)


_BN = N_PAD // 16


def _tc_tables1(xp, wcat3):
    def body(x_ref, w_ref, o36_ref, oD_ref):
        h = jnp.dot(x_ref[...], w_ref[0], preferred_element_type=jnp.float32)
        o36_ref[0] = h[:, :W36]
        oD_ref[0] = h[:, W36:]

    return pl.pallas_call(
        body,
        grid=(NC, N_PAD // _BN),
        in_specs=[pl.BlockSpec((_BN, D_IN), lambda c, i: (i, 0)),
                  pl.BlockSpec((1, D_IN, W36 + W4), lambda c, i: (c, 0, 0))],
        out_specs=[pl.BlockSpec((1, _BN, W36), lambda c, i: (c, i, 0)),
                   pl.BlockSpec((1, _BN, W4), lambda c, i: (c, i, 0))],
        out_shape=[jax.ShapeDtypeStruct((NC, N_PAD, W36), jnp.float32),
                   jax.ShapeDtypeStruct((NC, N_PAD, W4), jnp.float32)],
    )(xp, wcat3)


_BM = 1000


def _tc_mid(part1, b1row, rrep, m8, adv):
    def body(p_ref, b_ref, r_ref, m_ref, a_ref, t8_ref, ad_ref):
        num = jnp.concatenate(
            [p_ref[0, :, :HH * C1], p_ref[1, :, :HH * C1]], axis=1)
        den = jnp.concatenate(
            [p_ref[0, :, HH * C1:], p_ref[1, :, HH * C1:]], axis=1)
        denr = jnp.dot(den, r_ref[...], preferred_element_type=jnp.float32)
        out1 = num / denr + b_ref[...]
        t8_ref[...] = jnp.dot(out1, m_ref[...], preferred_element_type=jnp.float32)
        ad_ref[...] = jnp.dot(out1, a_ref[...], preferred_element_type=jnp.float32)

    return pl.pallas_call(
        body,
        grid=(N // _BM,),
        in_specs=[pl.BlockSpec((NC, _BM, W36), lambda i: (0, i, 0)),
                  pl.BlockSpec((1, H1 * C1), lambda i: (0, 0)),
                  pl.BlockSpec((H1, H1 * C1), lambda i: (0, 0)),
                  pl.BlockSpec((H1 * C1, W2_8), lambda i: (0, 0)),
                  pl.BlockSpec((H1 * C1, 1), lambda i: (0, 0))],
        out_specs=[pl.BlockSpec((_BM, W2_8), lambda i: (i, 0)),
                   pl.BlockSpec((_BM, 1), lambda i: (i, 0))],
        out_shape=[jax.ShapeDtypeStruct((N, W2_8), jnp.float32),
                   jax.ShapeDtypeStruct((N, 1), jnp.float32)],
    )(part1, b1row, rrep, m8, adv)


def _tc_final(part2, b2row):
    def body(p_ref, b_ref, o_ref):
        num = p_ref[0, :, :C2] + p_ref[1, :, :C2]
        den = p_ref[0, :, C2:] + p_ref[1, :, C2:]
        o = num / den + b_ref[...]
        m = jnp.max(o, axis=1, keepdims=True)
        ex = jnp.exp(o - m)
        o_ref[...] = (o - m) - jnp.log(jnp.sum(ex, axis=1, keepdims=True))

    return pl.pallas_call(
        body,
        grid=(N // _BM,),
        in_specs=[pl.BlockSpec((NC, _BM, W2_8), lambda i: (0, i, 0)),
                  pl.BlockSpec((1, C2), lambda i: (0, 0))],
        out_specs=pl.BlockSpec((_BM, C2), lambda i: (i, 0)),
        out_shape=jax.ShapeDtypeStruct((N, C2), jnp.float32),
    )(part2, b2row)


def _build_idx(src, dst, nsplit, nouter):
    idx = jnp.stack([src.reshape(nsplit, nouter, B, K),
                     dst.reshape(nsplit, nouter, B, K)], axis=2)
    idx = jnp.concatenate(
        [idx, jnp.zeros((nsplit, 1, 2, B, K), jnp.int32)], axis=1)
    return idx.reshape(nsplit * (nouter + 1) * 2 * B, K)


def kernel(x, edge_index, W1, att_src1, att_dst1, b1, W2, att_src2, att_dst2, b2):
    W1r = W1.reshape(D_IN, H1, C1)
    wsrc1 = jnp.einsum("dhc,hc->dh", W1r, att_src1[0])
    wdst1 = jnp.einsum("dhc,hc->dh", W1r, att_dst1[0])
    wcat3 = jnp.stack([
        jnp.concatenate([W1[:, :HH * C1], wsrc1[:, :HH], wdst1[:, :HH]], axis=1),
        jnp.concatenate([W1[:, HH * C1:], wsrc1[:, HH:], wdst1[:, HH:]], axis=1),
    ])

    m8 = jnp.concatenate([W2, (W2 @ att_src2[0, 0])[:, None]], axis=1)
    adv = (W2 @ att_dst2[0, 0])[:, None]
    rrep = jnp.repeat(jnp.eye(H1, dtype=jnp.float32), C1, axis=1)
    b1row = b1.reshape(1, H1 * C1)
    b2row = b2.reshape(1, C2)

    e_in = edge_index.shape[1]
    e_total = e_in + N
    loops = jnp.arange(N, dtype=jnp.int32)

    def padded(nsplit):
        nouter = -(-e_total // (nsplit * B * K))
        nouter += nouter % 2
        e_pad = nsplit * nouter * B * K
        padz = jnp.zeros((e_pad - e_total,), jnp.int32)
        s = jnp.concatenate([edge_index[0].astype(jnp.int32), loops, padz])
        d = jnp.concatenate([edge_index[1].astype(jnp.int32), loops, padz])
        return _build_idx(s, d, nsplit, nouter), nouter

    idx16, nouter16 = padded(NS)
    idx32, nouter32 = padded(NW)

    xp = jnp.concatenate([x, jnp.zeros((N_PAD - N, D_IN), jnp.float32)])
    zeros36 = jnp.zeros((N_PAD, W36), jnp.float32)
    zeros8 = jnp.zeros((N_PAD, W2_8), jnp.float32)

    t36, tD = _tc_tables1(xp, wcat3)
    part1 = _sc_edge_pass_l1(t36, tD, idx16, zeros36, nouter16, e_total)
    table8, ad2 = _tc_mid(part1, b1row, rrep, m8, adv)
    part2 = _sc_edge_pass_l2(table8, ad2.reshape(N), idx32, zeros8, nouter32, e_total)
    return _tc_final(part2, b2row)


# speedup vs baseline: 1.4515x; 1.0173x over previous
import functools

import jax
import jax.numpy as jnp
from jax import lax
from jax.experimental import pallas as pl
from jax.experimental.pallas import tpu as pltpu
from jax.experimental.pallas import tpu_sc as plsc

N = 10000
D_IN = 128
H1, C1 = 8, 8
H2, C2 = 1, 7

NC, NS, LANES = 2, 16, 16
NW = NC * NS
K = 128
B = 2
N_PAD = 10240
RPS = N_PAD // NS

HH = H1 // NC
W36 = 40
W4 = 8
W2_8 = C2 + 1

_MESH = plsc.VectorSubcoreMesh(
    core_axis_name="c", subcore_axis_name="s", num_cores=NC, num_subcores=NS)
_SC_PARAMS = pltpu.CompilerParams(
    needs_layout_passes=False, use_tc_tiling_on_sc=False)


def _sc_edge_pass_l1(t36, tD, idx, zeros36, nouter, e_total):
    T = nouter * B * K

    @functools.partial(
        pl.kernel,
        out_type=jax.ShapeDtypeStruct((NC, N_PAD, W36), jnp.float32),
        mesh=_MESH,
        compiler_params=_SC_PARAMS,
        scratch_types=[
            pltpu.VMEM_SHARED((N_PAD, W36), jnp.float32),
            pltpu.VMEM_SHARED((N_PAD, W36), jnp.float32),
            pltpu.VMEM_SHARED((N_PAD, W4), jnp.float32),
            [pltpu.VMEM((2 * B, K), jnp.int32)] * 2,
            [pltpu.VMEM((B * K, W36), jnp.float32)] * 2,
            [pltpu.VMEM((B * K, W4), jnp.float32)] * 2,
            [pltpu.VMEM((B * K, W36), jnp.float32)] * 2,
            pltpu.SemaphoreType.DMA,
            pltpu.SemaphoreType.DMA,
            pltpu.SemaphoreType.DMA,
        ],
    )
    def k(t36_hbm, tD_hbm, idx_hbm, z_hbm, part_hbm,
          accum, tsh, adsh, idxv, rowsv, rowsdv, outv, gsem, g2sem, ssem):
        cid = lax.axis_index("c")
        sid = lax.axis_index("s")
        r0 = sid * RPS
        pltpu.sync_copy(z_hbm.at[pl.ds(r0, RPS)], accum.at[pl.ds(r0, RPS)])
        pltpu.sync_copy(t36_hbm.at[cid, pl.ds(r0, RPS)], tsh.at[pl.ds(r0, RPS)])
        pltpu.sync_copy(tD_hbm.at[cid, pl.ds(r0, RPS)], adsh.at[pl.ds(r0, RPS)])
        plsc.subcore_barrier()

        iota = lax.iota(jnp.int32, LANES)

        def load_idx(it, buf):
            r = (sid * (nouter + 1) + it) * (2 * B)
            pltpu.sync_copy(idx_hbm.at[pl.ds(r, 2 * B)], idxv[buf])

        def fire_gathers(buf):
            ds = []
            for j in range(B):
                ds.append(pltpu.async_copy(
                    tsh.at[idxv[buf].at[j]],
                    rowsv[buf].at[pl.ds(j * K, K)], gsem))
                ds.append(pltpu.async_copy(
                    adsh.at[idxv[buf].at[B + j]],
                    rowsdv[buf].at[pl.ds(j * K, K)], g2sem))
            return ds

        def compute(it, buf):
            rv, rdv, ov = rowsv[buf], rowsdv[buf], outv[buf]
            ebase = sid * T + it * (B * K)

            def group(g, carry2):
                rows = g * LANES + iota
                fmask = jnp.where((ebase + g * LANES + iota) < e_total, 1.0, 0.0)
                for hd in range(HH):
                    colw = jnp.full((LANES,), HH * C1 + hd, jnp.int32)
                    a_s = plsc.load_gather(rv, [rows, colw])
                    a_d = plsc.load_gather(rdv, [rows, jnp.full((LANES,), hd, jnp.int32)])
                    alpha = a_s + a_d
                    alpha = jnp.where(alpha >= 0, alpha, 0.2 * alpha)
                    w = jnp.exp(alpha) * fmask
                    plsc.store_scatter(ov, [rows, colw], w)
                    for c in range(C1):
                        col = jnp.full((LANES,), hd * C1 + c, jnp.int32)
                        hv = plsc.load_gather(rv, [rows, col])
                        plsc.store_scatter(ov, [rows, col], w * hv)
                return carry2

            lax.fori_loop(0, B * K // LANES, group, 0)

        def fire_scatters(buf):
            return [pltpu.async_copy(
                outv[buf].at[pl.ds(j * K, K)],
                accum.at[idxv[buf].at[B + j]], ssem, add=True)
                for j in range(B)]

        load_idx(0, 0)
        for d in fire_gathers(0):
            d.wait()

        def pair(ip, carry):
            for cur in range(2):
                it = ip * 2 + cur
                nxt = 1 - cur
                load_idx(it + 1, nxt)
                gds = fire_gathers(nxt)
                compute(it, cur)
                sds = fire_scatters(cur)
                for d in gds:
                    d.wait()
                for d in sds:
                    d.wait()
            return carry

        lax.fori_loop(0, nouter // 2, pair, 0)
        plsc.subcore_barrier()
        pltpu.sync_copy(accum.at[pl.ds(r0, RPS)], part_hbm.at[cid, pl.ds(r0, RPS)])

    return k(t36, tD, idx, zeros36)


def _sc_edge_pass_l2(table8, ad2, idx, zeros8, nouter, e_total):
    T = nouter * B * K

    @functools.partial(
        pl.kernel,
        out_type=jax.ShapeDtypeStruct((NC, N_PAD, W2_8), jnp.float32),
        mesh=_MESH,
        compiler_params=_SC_PARAMS,
        scratch_types=[
            pltpu.VMEM_SHARED((N_PAD, W2_8), jnp.float32),
            pltpu.VMEM((N, W2_8), jnp.float32),
            pltpu.VMEM((N,), jnp.float32),
            [pltpu.VMEM((2 * B, K), jnp.int32)] * 2,
            [pltpu.VMEM((B * K, W2_8), jnp.float32)] * 2,
            pltpu.SemaphoreType.DMA,
        ],
    )
    def k(t8_hbm, ad_hbm, idx_hbm, z_hbm, part_hbm,
          accum, t8buf, adbuf, idxv, outv, ssem):
        cid = lax.axis_index("c")
        sid = lax.axis_index("s")
        wid = sid * NC + cid
        r0 = sid * RPS
        pltpu.sync_copy(z_hbm.at[pl.ds(r0, RPS)], accum.at[pl.ds(r0, RPS)])
        pltpu.sync_copy(t8_hbm, t8buf)
        pltpu.sync_copy(ad_hbm, adbuf)
        plsc.subcore_barrier()

        iota = lax.iota(jnp.int32, LANES)

        def load_idx(it, buf):
            r = (wid * (nouter + 1) + it) * (2 * B)
            pltpu.sync_copy(idx_hbm.at[pl.ds(r, 2 * B)], idxv[buf])

        def compute(it, buf):
            ov, iv = outv[buf], idxv[buf]
            ebase = wid * T + it * (B * K)

            def group(g, carry2):
                rows = g * LANES + iota
                fmask = jnp.where((ebase + g * LANES + iota) < e_total, 1.0, 0.0)
                j = g // (K // LANES)
                gk = g % (K // LANES)
                svals = iv[j, pl.ds(gk * LANES, LANES)]
                dvals = iv[B + j, pl.ds(gk * LANES, LANES)]
                colw = jnp.full((LANES,), C2, jnp.int32)
                a_s = plsc.load_gather(t8buf, [svals, colw])
                a_d = plsc.load_gather(adbuf, [dvals])
                alpha = a_s + a_d
                alpha = jnp.where(alpha >= 0, alpha, 0.2 * alpha)
                w = jnp.exp(alpha) * fmask
                plsc.store_scatter(ov, [rows, colw], w)
                for c in range(C2):
                    col = jnp.full((LANES,), c, jnp.int32)
                    hv = plsc.load_gather(t8buf, [svals, col])
                    plsc.store_scatter(ov, [rows, col], w * hv)
                return carry2

            lax.fori_loop(0, B * K // LANES, group, 0)

        def fire_scatters(buf):
            return [pltpu.async_copy(
                outv[buf].at[pl.ds(j * K, K)],
                accum.at[idxv[buf].at[B + j]], ssem, add=True)
                for j in range(B)]

        load_idx(0, 0)

        def pair(ip, carry):
            for cur in range(2):
                it = ip * 2 + cur
                nxt = 1 - cur
                load_idx(it + 1, nxt)
                compute(it, cur)
                sds = fire_scatters(cur)
                for d in sds:
                    d.wait()
            return carry

        lax.fori_loop(0, nouter // 2, pair, 0)
        plsc.subcore_barrier()
        pltpu.sync_copy(accum.at[pl.ds(r0, RPS)], part_hbm.at[cid, pl.ds(r0, RPS)])

    return k(table8, ad2, idx, zeros8)


_BN = N_PAD // 16


def _tc_tables1(xp, wcat3):
    def body(x_ref, w_ref, o36_ref, oD_ref):
        h = jnp.dot(x_ref[...], w_ref[0], preferred_element_type=jnp.float32)
        o36_ref[0] = h[:, :W36]
        oD_ref[0] = h[:, W36:]

    return pl.pallas_call(
        body,
        grid=(NC, N_PAD // _BN),
        in_specs=[pl.BlockSpec((_BN, D_IN), lambda c, i: (i, 0)),
                  pl.BlockSpec((1, D_IN, W36 + W4), lambda c, i: (c, 0, 0))],
        out_specs=[pl.BlockSpec((1, _BN, W36), lambda c, i: (c, i, 0)),
                   pl.BlockSpec((1, _BN, W4), lambda c, i: (c, i, 0))],
        out_shape=[jax.ShapeDtypeStruct((NC, N_PAD, W36), jnp.float32),
                   jax.ShapeDtypeStruct((NC, N_PAD, W4), jnp.float32)],
    )(xp, wcat3)


_BM = 1000


def _tc_mid(part1, b1row, rrep, m8, adv):
    def body(p_ref, b_ref, r_ref, m_ref, a_ref, t8_ref, ad_ref):
        num = jnp.concatenate(
            [p_ref[0, :, :HH * C1], p_ref[1, :, :HH * C1]], axis=1)
        den = jnp.concatenate(
            [p_ref[0, :, HH * C1:HH * C1 + HH], p_ref[1, :, HH * C1:HH * C1 + HH]],
            axis=1)
        denr = jnp.dot(den, r_ref[...], preferred_element_type=jnp.float32)
        out1 = num / denr + b_ref[...]
        t8_ref[...] = jnp.dot(out1, m_ref[...], preferred_element_type=jnp.float32)
        ad_ref[...] = jnp.dot(out1, a_ref[...], preferred_element_type=jnp.float32)

    return pl.pallas_call(
        body,
        grid=(N // _BM,),
        in_specs=[pl.BlockSpec((NC, _BM, W36), lambda i: (0, i, 0)),
                  pl.BlockSpec((1, H1 * C1), lambda i: (0, 0)),
                  pl.BlockSpec((H1, H1 * C1), lambda i: (0, 0)),
                  pl.BlockSpec((H1 * C1, W2_8), lambda i: (0, 0)),
                  pl.BlockSpec((H1 * C1, 1), lambda i: (0, 0))],
        out_specs=[pl.BlockSpec((_BM, W2_8), lambda i: (i, 0)),
                   pl.BlockSpec((_BM, 1), lambda i: (i, 0))],
        out_shape=[jax.ShapeDtypeStruct((N, W2_8), jnp.float32),
                   jax.ShapeDtypeStruct((N, 1), jnp.float32)],
    )(part1, b1row, rrep, m8, adv)


def _tc_final(part2, b2row):
    def body(p_ref, b_ref, o_ref):
        num = p_ref[0, :, :C2] + p_ref[1, :, :C2]
        den = p_ref[0, :, C2:] + p_ref[1, :, C2:]
        o = num / den + b_ref[...]
        m = jnp.max(o, axis=1, keepdims=True)
        ex = jnp.exp(o - m)
        o_ref[...] = (o - m) - jnp.log(jnp.sum(ex, axis=1, keepdims=True))

    return pl.pallas_call(
        body,
        grid=(N // _BM,),
        in_specs=[pl.BlockSpec((NC, _BM, W2_8), lambda i: (0, i, 0)),
                  pl.BlockSpec((1, C2), lambda i: (0, 0))],
        out_specs=pl.BlockSpec((_BM, C2), lambda i: (i, 0)),
        out_shape=jax.ShapeDtypeStruct((N, C2), jnp.float32),
    )(part2, b2row)


def _build_idx(src, dst, nsplit, nouter):
    idx = jnp.stack([src.reshape(nsplit, nouter, B, K),
                     dst.reshape(nsplit, nouter, B, K)], axis=2)
    idx = jnp.concatenate(
        [idx, jnp.zeros((nsplit, 1, 2, B, K), jnp.int32)], axis=1)
    return idx.reshape(nsplit * (nouter + 1) * 2 * B, K)


def kernel(x, edge_index, W1, att_src1, att_dst1, b1, W2, att_src2, att_dst2, b2):
    W1r = W1.reshape(D_IN, H1, C1)
    wsrc1 = jnp.einsum("dhc,hc->dh", W1r, att_src1[0])
    wdst1 = jnp.einsum("dhc,hc->dh", W1r, att_dst1[0])
    zc = jnp.zeros((D_IN, W36 - HH * C1 - HH), jnp.float32)
    zd = jnp.zeros((D_IN, W4 - HH), jnp.float32)
    wcat3 = jnp.stack([
        jnp.concatenate([W1[:, :HH * C1], wsrc1[:, :HH], zc,
                         wdst1[:, :HH], zd], axis=1),
        jnp.concatenate([W1[:, HH * C1:], wsrc1[:, HH:], zc,
                         wdst1[:, HH:], zd], axis=1),
    ])

    m8 = jnp.concatenate([W2, (W2 @ att_src2[0, 0])[:, None]], axis=1)
    adv = (W2 @ att_dst2[0, 0])[:, None]
    rrep = jnp.repeat(jnp.eye(H1, dtype=jnp.float32), C1, axis=1)
    b1row = b1.reshape(1, H1 * C1)
    b2row = b2.reshape(1, C2)

    e_in = edge_index.shape[1]
    e_total = e_in + N
    loops = jnp.arange(N, dtype=jnp.int32)

    def padded(nsplit):
        nouter = -(-e_total // (nsplit * B * K))
        nouter += nouter % 2
        e_pad = nsplit * nouter * B * K
        padz = jnp.zeros((e_pad - e_total,), jnp.int32)
        s = jnp.concatenate([edge_index[0].astype(jnp.int32), loops, padz])
        d = jnp.concatenate([edge_index[1].astype(jnp.int32), loops, padz])
        return _build_idx(s, d, nsplit, nouter), nouter

    idx16, nouter16 = padded(NS)
    idx32, nouter32 = padded(NW)

    xp = jnp.concatenate([x, jnp.zeros((N_PAD - N, D_IN), jnp.float32)])
    zeros36 = jnp.zeros((N_PAD, W36), jnp.float32)
    zeros8 = jnp.zeros((N_PAD, W2_8), jnp.float32)

    t36, tD = _tc_tables1(xp, wcat3)
    part1 = _sc_edge_pass_l1(t36, tD, idx16, zeros36, nouter16, e_total)
    table8, ad2 = _tc_mid(part1, b1row, rrep, m8, adv)
    part2 = _sc_edge_pass_l2(table8, ad2.reshape(N), idx32, zeros8, nouter32, e_total)
    return _tc_final(part2, b2row)
